# Initial kernel scaffold; baseline (speedup 1.0000x reference)
import jax
import jax.numpy as jnp
from jax.experimental import pallas as pl


def kernel(node_feat, edge_index, self_feat, W1, b1, W2, b2, Wf1, bf1, Wf2, bf2):
    raise NotImplementedError("write your pallas kernel here")


# speedup vs baseline: 6.3008x; 6.3008x over previous
import functools

import jax
import jax.numpy as jnp
from jax import lax
from jax.experimental import pallas as pl
from jax.experimental.pallas import tpu as pltpu
from jax.experimental.pallas import tpu_sc as plsc

N_NODES = 10000
N_EDGES = 320000
D1P = 128
D2P = 128
CHK = 128
NC = 2
NS = 16
NW = NC * NS
NCHUNK = N_EDGES // CHK
CPW = (NCHUNK + NW - 1) // NW
NP = 10240
ROWS_PER_TILE = NP // NS
ZR = 128
BLK = 1000


def _seg_sum_sc(table, srcr, dstr, d):
    mesh = plsc.VectorSubcoreMesh(core_axis_name="c", subcore_axis_name="s")

    @functools.partial(
        pl.kernel,
        out_type=jax.ShapeDtypeStruct((NC, NP, d), jnp.float32),
        mesh=mesh,
        scratch_types=[
            pltpu.VMEM((CHK,), jnp.int32),
            pltpu.VMEM((CHK,), jnp.int32),
            pltpu.VMEM((CHK, d), jnp.float32),
            pltpu.VMEM((ZR, d), jnp.float32),
            pltpu.VMEM_SHARED((NP, d), jnp.float32),
            pltpu.SemaphoreType.DMA,
        ],
    )
    def k(table_hbm, src_hbm, dst_hbm, out_hbm, sidx, didx, rows, zbuf, acc, sem):
        cid = lax.axis_index("c")
        sid = lax.axis_index("s")
        wid = sid * NC + cid

        @pl.loop(0, ZR)
        def _zero(r):
            for j in range(d // 16):
                zbuf[r, pl.ds(j * 16, 16)] = jnp.zeros((16,), jnp.float32)

        row0 = sid * ROWS_PER_TILE
        for b in range(ROWS_PER_TILE // ZR):
            pltpu.sync_copy(zbuf, acc.at[pl.ds(row0 + b * ZR, ZR), :])
        plsc.subcore_barrier()

        @pl.loop(0, CPW)
        def _chunk(kk):
            c = wid + kk * NW

            @pl.when(c < NCHUNK)
            def _():
                pltpu.sync_copy(src_hbm.at[c], sidx)
                pltpu.sync_copy(dst_hbm.at[c], didx)
                pltpu.async_copy(table_hbm.at[sidx], rows, sem).wait()
                pltpu.sync_copy(rows, acc.at[didx], add=True)

        plsc.subcore_barrier()
        pltpu.sync_copy(acc.at[pl.ds(row0, ROWS_PER_TILE), :],
                        out_hbm.at[cid, pl.ds(row0, ROWS_PER_TILE), :])

    return k(table, srcr, dstr)


def _mm1_body(x_ref, w_ref, b_ref, o_ref):
    o_ref[...] = (
        jnp.dot(x_ref[...], w_ref[...], preferred_element_type=jnp.float32)
        + b_ref[...]
    )


def _layer2_body(sp_ref, t1_ref, b_ref, w_ref, t2_ref, deg_ref):
    s = sp_ref[0] + sp_ref[1]
    deg = s[:, 100:101]
    t1 = t1_ref[...]
    mean = jnp.where(deg > 0, s / jnp.maximum(deg, 1.0), t1)
    h1 = jnp.maximum(mean + b_ref[...], 0.0)
    t2_ref[...] = jnp.dot(h1, w_ref[...], preferred_element_type=jnp.float32)
    deg_ref[...] = deg


def _head_body(sp_ref, t2_ref, deg_ref, b2_ref, self_ref, wf1_ref, bf1_ref,
               wf2_ref, bf2_ref, o_ref, acc_ref):
    i = pl.program_id(0)
    s = sp_ref[0] + sp_ref[1]
    deg = deg_ref[...]
    mean = jnp.where(deg > 0, s / jnp.maximum(deg, 1.0), t2_ref[...])
    h2 = jnp.maximum(mean + b2_ref[...], 0.0)
    csum = jnp.sum(h2, axis=0, keepdims=True)

    @pl.when(i == 0)
    def _():
        acc_ref[0:1, 0:D2P] = csum

    @pl.when(i > 0)
    def _():
        acc_ref[0:1, 0:D2P] += csum

    @pl.when(i == pl.num_programs(0) - 1)
    def _():
        hg = acc_ref[0:1, 0:20] * (1.0 / N_NODES)
        fused = jnp.concatenate([hg, self_ref[...]], axis=1)
        o1 = jnp.maximum(
            jnp.dot(fused, wf1_ref[...], preferred_element_type=jnp.float32)
            + bf1_ref[...], 0.0)
        o_ref[...] = (
            jnp.dot(o1, wf2_ref[...], preferred_element_type=jnp.float32)
            + bf2_ref[...]
        )


def kernel(node_feat, edge_index, self_feat, W1, b1, W2, b2, Wf1, bf1, Wf2, bf2):
    src = edge_index[0].reshape(NCHUNK, CHK)
    dst = edge_index[1].reshape(NCHUNK, CHK)

    W1Tp = jnp.zeros((128, D1P), jnp.float32).at[:, :100].set(W1.T)
    e1 = jnp.zeros((1, D1P), jnp.float32).at[0, 100].set(1.0)
    b1p = jnp.zeros((1, D1P), jnp.float32).at[0, :100].set(b1)
    W2Tp = jnp.zeros((D1P, D2P), jnp.float32).at[:100, :20].set(W2.T)
    b2p = jnp.zeros((1, D2P), jnp.float32).at[0, :20].set(b2)
    Wf1T = Wf1.T
    bf1r = bf1.reshape(1, 10)
    Wf2T = Wf2.T
    bf2r = bf2.reshape(1, 8)

    ngrid = N_NODES // BLK

    t1 = pl.pallas_call(
        _mm1_body,
        grid=(ngrid,),
        in_specs=[
            pl.BlockSpec((BLK, 128), lambda i: (i, 0)),
            pl.BlockSpec((128, D1P), lambda i: (0, 0)),
            pl.BlockSpec((1, D1P), lambda i: (0, 0)),
        ],
        out_specs=pl.BlockSpec((BLK, D1P), lambda i: (i, 0)),
        out_shape=jax.ShapeDtypeStruct((N_NODES, D1P), jnp.float32),
    )(node_feat, W1Tp, e1)

    s1p = _seg_sum_sc(t1, src, dst, D1P)

    t2, deg = pl.pallas_call(
        _layer2_body,
        grid=(ngrid,),
        in_specs=[
            pl.BlockSpec((NC, BLK, D1P), lambda i: (0, i, 0)),
            pl.BlockSpec((BLK, D1P), lambda i: (i, 0)),
            pl.BlockSpec((1, D1P), lambda i: (0, 0)),
            pl.BlockSpec((D1P, D2P), lambda i: (0, 0)),
        ],
        out_specs=[
            pl.BlockSpec((BLK, D2P), lambda i: (i, 0)),
            pl.BlockSpec((BLK, 1), lambda i: (i, 0)),
        ],
        out_shape=[
            jax.ShapeDtypeStruct((N_NODES, D2P), jnp.float32),
            jax.ShapeDtypeStruct((N_NODES, 1), jnp.float32),
        ],
    )(s1p, t1, b1p, W2Tp)

    s2p = _seg_sum_sc(t2, src, dst, D2P)

    out = pl.pallas_call(
        _head_body,
        grid=(ngrid,),
        in_specs=[
            pl.BlockSpec((NC, BLK, D2P), lambda i: (0, i, 0)),
            pl.BlockSpec((BLK, D2P), lambda i: (i, 0)),
            pl.BlockSpec((BLK, 1), lambda i: (i, 0)),
            pl.BlockSpec((1, D2P), lambda i: (0, 0)),
            pl.BlockSpec((1, 16), lambda i: (0, 0)),
            pl.BlockSpec((36, 10), lambda i: (0, 0)),
            pl.BlockSpec((1, 10), lambda i: (0, 0)),
            pl.BlockSpec((10, 8), lambda i: (0, 0)),
            pl.BlockSpec((1, 8), lambda i: (0, 0)),
        ],
        out_specs=pl.BlockSpec((1, 8), lambda i: (0, 0)),
        out_shape=jax.ShapeDtypeStruct((1, 8), jnp.float32),
        scratch_shapes=[pltpu.VMEM((8, 128), jnp.float32)],
    )(s2p, t2, deg, b2p, self_feat, Wf1T, bf1r, Wf2T, bf2r)

    return out


# speedup vs baseline: 8.3692x; 1.3283x over previous
import functools

import jax
import jax.numpy as jnp
from jax import lax
from jax.experimental import pallas as pl
from jax.experimental.pallas import tpu as pltpu
from jax.experimental.pallas import tpu_sc as plsc

N_NODES = 10000
N_EDGES = 320000
D1P = 128
D2P = 128
CHK = 128
NC = 2
NS = 16
NW = NC * NS
NCHUNK = N_EDGES // CHK
CPW = (NCHUNK + NW - 1) // NW
NP = 10240
HNP = NP // NC
ACC_R = 5248
ZR = 82
NITER = 160
BLK = 1000


def _seg_sum_sc(table, srcr, dstr):
    mesh = plsc.VectorSubcoreMesh(core_axis_name="c", subcore_axis_name="s")
    NBUF = 4

    @functools.partial(
        pl.kernel,
        out_type=jax.ShapeDtypeStruct((NP, 128), jnp.float32),
        mesh=mesh,
        scratch_types=(
            [pltpu.VMEM((CHK,), jnp.int32) for _ in range(NBUF)]
            + [pltpu.VMEM((CHK,), jnp.int32) for _ in range(NBUF)]
            + [pltpu.VMEM((CHK, 128), jnp.float32) for _ in range(NBUF)]
            + [pltpu.VMEM((ZR, 128), jnp.float32)]
            + [pltpu.VMEM_SHARED((ACC_R, 128), jnp.float32)]
            + [pltpu.SemaphoreType.DMA for _ in range(3 * NBUF)]
        ),
    )
    def k(table_hbm, src_hbm, dst_hbm, out_hbm, *refs):
        sidx = refs[0:NBUF]
        didx = refs[NBUF:2 * NBUF]
        rows = refs[2 * NBUF:3 * NBUF]
        zbuf = refs[3 * NBUF]
        acc = refs[3 * NBUF + 1]
        isem = refs[3 * NBUF + 2:3 * NBUF + 2 + NBUF]
        gsem = refs[3 * NBUF + 2 + NBUF:3 * NBUF + 2 + 2 * NBUF]
        ssem = refs[3 * NBUF + 2 + 2 * NBUF:3 * NBUF + 2 + 3 * NBUF]

        cid = lax.axis_index("c")
        sid = lax.axis_index("s")
        base = cid * HNP

        @pl.loop(0, ZR)
        def _zero(r):
            for j in range(128 // 16):
                zbuf[r, pl.ds(j * 16, 16)] = jnp.zeros((16,), jnp.float32)

        row0z = sid * (ACC_R // NS)
        for b in range(ACC_R // NS // ZR):
            pltpu.sync_copy(zbuf, acc.at[pl.ds(row0z + b * ZR, ZR), :])
        plsc.subcore_barrier()

        def c_of(i):
            return sid + i * NS

        def ok(i):
            return (i >= 0) & (c_of(i) < NCHUNK)

        def issue_idx(i, b):
            @pl.when(ok(i))
            def _():
                pltpu.async_copy(src_hbm.at[c_of(i)], sidx[b], isem[b])
                pltpu.async_copy(dst_hbm.at[c_of(i)], didx[b], isem[b])

        def wait_idx_and_localize(i, b):
            @pl.when(ok(i))
            def _():
                pltpu.make_async_copy(src_hbm.at[0], sidx[b], isem[b]).wait()
                pltpu.make_async_copy(dst_hbm.at[0], didx[b], isem[b]).wait()
                for j in range(CHK // 16):
                    v = didx[b][pl.ds(j * 16, 16)] - base
                    m = (v >= 0) & (v < HNP)
                    dump = (jnp.zeros((16,), jnp.int32)
                            + (HNP + sid * 8 + (j % 8)))
                    didx[b][pl.ds(j * 16, 16)] = jnp.where(m, v, dump)

        def issue_gather(i, b):
            @pl.when(ok(i))
            def _():
                pltpu.async_copy(table_hbm.at[sidx[b]], rows[b], gsem[b])

        def wait_gather(i, b):
            @pl.when(ok(i))
            def _():
                pltpu.make_async_copy(table_hbm.at[sidx[b]], rows[b],
                                      gsem[b]).wait()

        def issue_scatter(i, b):
            @pl.when(ok(i))
            def _():
                pltpu.async_copy(rows[b], acc.at[didx[b]], ssem[b], add=True)

        def wait_scatter(i, b):
            @pl.when(ok(i))
            def _():
                pltpu.make_async_copy(rows[b], acc.at[didx[b]], ssem[b]).wait()

        issue_idx(jnp.int32(0), 0)

        @pl.loop(0, NITER, step=NBUF)
        def _steady(k0):
            for b in range(NBUF):
                i = k0 + b
                wait_scatter(i - 3, (b + 1) % NBUF)
                issue_idx(i + 1, (b + 1) % NBUF)
                wait_idx_and_localize(i, b)
                issue_gather(i, b)
                wait_gather(i - 1, (b - 1) % NBUF)
                issue_scatter(i - 1, (b - 1) % NBUF)

        last = jnp.int32(NITER - 1)
        wait_scatter(last - 2, (NITER - 3) % NBUF)
        wait_scatter(last - 1, (NITER - 2) % NBUF)

        plsc.subcore_barrier()
        row0 = sid * (HNP // NS)
        pltpu.sync_copy(acc.at[pl.ds(row0, HNP // NS), :],
                        out_hbm.at[pl.ds(base + row0, HNP // NS), :])

    return k(table, srcr, dstr)


def _mm1_body(x_ref, w_ref, b_ref, o_ref):
    o_ref[...] = (
        jnp.dot(x_ref[...], w_ref[...], preferred_element_type=jnp.float32)
        + b_ref[...]
    )


def _layer2_body(sp_ref, t1_ref, b_ref, w_ref, t2_ref, deg_ref):
    s = sp_ref[...]
    deg = s[:, 100:101]
    s = s[:, :112]
    t1 = t1_ref[:, :112]
    mean = jnp.where(deg > 0, s / jnp.maximum(deg, 1.0), t1)
    h1 = jnp.maximum(mean + b_ref[...], 0.0)
    t2_ref[...] = jnp.dot(h1, w_ref[...], preferred_element_type=jnp.float32)
    deg_ref[...] = deg


def _head_body(sp_ref, t2_ref, deg_ref, b2_ref, self_ref, wf1_ref, bf1_ref,
               wf2_ref, bf2_ref, o_ref, acc_ref):
    i = pl.program_id(0)
    s = sp_ref[:, :32]
    deg = deg_ref[...]
    mean = jnp.where(deg > 0, s / jnp.maximum(deg, 1.0), t2_ref[:, :32])
    h2 = jnp.maximum(mean + b2_ref[...], 0.0)
    csum = jnp.sum(h2, axis=0, keepdims=True)

    @pl.when(i == 0)
    def _():
        acc_ref[0:1, 0:32] = csum

    @pl.when(i > 0)
    def _():
        acc_ref[0:1, 0:32] += csum

    @pl.when(i == pl.num_programs(0) - 1)
    def _():
        hg = acc_ref[0:1, 0:20] * (1.0 / N_NODES)
        fused = jnp.concatenate([hg, self_ref[...]], axis=1)
        o1 = jnp.maximum(
            jnp.dot(fused, wf1_ref[...], preferred_element_type=jnp.float32)
            + bf1_ref[...], 0.0)
        o_ref[...] = (
            jnp.dot(o1, wf2_ref[...], preferred_element_type=jnp.float32)
            + bf2_ref[...]
        )


def kernel(node_feat, edge_index, self_feat, W1, b1, W2, b2, Wf1, bf1, Wf2, bf2):
    src = edge_index[0].reshape(NCHUNK, CHK)
    dst = edge_index[1].reshape(NCHUNK, CHK)

    W1Tp = jnp.zeros((128, D1P), jnp.float32).at[:, :100].set(W1.T)
    e1 = jnp.zeros((1, D1P), jnp.float32).at[0, 100].set(1.0)
    b1p = jnp.zeros((1, 112), jnp.float32).at[0, :100].set(b1)
    W2Tp = jnp.zeros((112, D2P), jnp.float32).at[:100, :20].set(W2.T)
    b2p = jnp.zeros((1, 32), jnp.float32).at[0, :20].set(b2)
    Wf1T = Wf1.T
    bf1r = bf1.reshape(1, 10)
    Wf2T = Wf2.T
    bf2r = bf2.reshape(1, 8)

    ngrid = N_NODES // BLK

    t1 = pl.pallas_call(
        _mm1_body,
        grid=(ngrid,),
        in_specs=[
            pl.BlockSpec((BLK, 128), lambda i: (i, 0)),
            pl.BlockSpec((128, D1P), lambda i: (0, 0)),
            pl.BlockSpec((1, D1P), lambda i: (0, 0)),
        ],
        out_specs=pl.BlockSpec((BLK, D1P), lambda i: (i, 0)),
        out_shape=jax.ShapeDtypeStruct((N_NODES, D1P), jnp.float32),
    )(node_feat, W1Tp, e1)

    s1 = _seg_sum_sc(t1, src, dst)

    t2, deg = pl.pallas_call(
        _layer2_body,
        grid=(ngrid,),
        in_specs=[
            pl.BlockSpec((BLK, D1P), lambda i: (i, 0)),
            pl.BlockSpec((BLK, D1P), lambda i: (i, 0)),
            pl.BlockSpec((1, 112), lambda i: (0, 0)),
            pl.BlockSpec((112, D2P), lambda i: (0, 0)),
        ],
        out_specs=[
            pl.BlockSpec((BLK, D2P), lambda i: (i, 0)),
            pl.BlockSpec((BLK, 1), lambda i: (i, 0)),
        ],
        out_shape=[
            jax.ShapeDtypeStruct((N_NODES, D2P), jnp.float32),
            jax.ShapeDtypeStruct((N_NODES, 1), jnp.float32),
        ],
    )(s1, t1, b1p, W2Tp)

    s2 = _seg_sum_sc(t2, src, dst)

    out = pl.pallas_call(
        _head_body,
        grid=(ngrid,),
        in_specs=[
            pl.BlockSpec((BLK, D2P), lambda i: (i, 0)),
            pl.BlockSpec((BLK, D2P), lambda i: (i, 0)),
            pl.BlockSpec((BLK, 1), lambda i: (i, 0)),
            pl.BlockSpec((1, 32), lambda i: (0, 0)),
            pl.BlockSpec((1, 16), lambda i: (0, 0)),
            pl.BlockSpec((36, 10), lambda i: (0, 0)),
            pl.BlockSpec((1, 10), lambda i: (0, 0)),
            pl.BlockSpec((10, 8), lambda i: (0, 0)),
            pl.BlockSpec((1, 8), lambda i: (0, 0)),
        ],
        out_specs=pl.BlockSpec((1, 8), lambda i: (0, 0)),
        out_shape=jax.ShapeDtypeStruct((1, 8), jnp.float32),
        scratch_shapes=[pltpu.VMEM((8, 128), jnp.float32)],
    )(s2, t2, deg, b2p, self_feat, Wf1T, bf1r, Wf2T, bf2r)

    return out


# speedup vs baseline: 8.3913x; 1.0026x over previous
import functools

import jax
import jax.numpy as jnp
from jax import lax
from jax.experimental import pallas as pl
from jax.experimental.pallas import tpu as pltpu
from jax.experimental.pallas import tpu_sc as plsc

N_NODES = 10000
N_EDGES = 320000
D1P = 128
D2P = 128
CHK = 128
NC = 2
NS = 16
NW = NC * NS
NCHUNK = N_EDGES // CHK
CPW = (NCHUNK + NW - 1) // NW
NP = 10240
HNP = NP // NC
ACC_R = 5184
ZR = 27
NITER = 160
BLK = 1000


def _seg_sum_sc(table, srcr, dstr):
    mesh = plsc.VectorSubcoreMesh(core_axis_name="c", subcore_axis_name="s")
    NBUF = 5

    @functools.partial(
        pl.kernel,
        out_type=jax.ShapeDtypeStruct((NP, 128), jnp.float32),
        mesh=mesh,
        scratch_types=(
            [pltpu.VMEM((CHK,), jnp.int32) for _ in range(NBUF)]
            + [pltpu.VMEM((CHK,), jnp.int32) for _ in range(NBUF)]
            + [pltpu.VMEM((CHK, 128), jnp.float32) for _ in range(NBUF)]
            + [pltpu.VMEM((ZR, 128), jnp.float32)]
            + [pltpu.VMEM_SHARED((ACC_R, 128), jnp.float32)]
            + [pltpu.SemaphoreType.DMA for _ in range(3 * NBUF)]
        ),
    )
    def k(table_hbm, src_hbm, dst_hbm, out_hbm, *refs):
        sidx = refs[0:NBUF]
        didx = refs[NBUF:2 * NBUF]
        rows = refs[2 * NBUF:3 * NBUF]
        zbuf = refs[3 * NBUF]
        acc = refs[3 * NBUF + 1]
        isem = refs[3 * NBUF + 2:3 * NBUF + 2 + NBUF]
        gsem = refs[3 * NBUF + 2 + NBUF:3 * NBUF + 2 + 2 * NBUF]
        ssem = refs[3 * NBUF + 2 + 2 * NBUF:3 * NBUF + 2 + 3 * NBUF]

        cid = lax.axis_index("c")
        sid = lax.axis_index("s")
        base = cid * HNP

        @pl.loop(0, ZR)
        def _zero(r):
            for j in range(128 // 16):
                zbuf[r, pl.ds(j * 16, 16)] = jnp.zeros((16,), jnp.float32)

        row0z = sid * (ACC_R // NS)
        for b in range(ACC_R // NS // ZR):
            pltpu.sync_copy(zbuf, acc.at[pl.ds(row0z + b * ZR, ZR), :])
        plsc.subcore_barrier()

        def c_of(i):
            return sid + i * NS

        def ok(i):
            return (i >= 0) & (c_of(i) < NCHUNK)

        def issue_idx(i, b):
            @pl.when(ok(i))
            def _():
                pltpu.async_copy(src_hbm.at[c_of(i)], sidx[b], isem[b])
                pltpu.async_copy(dst_hbm.at[c_of(i)], didx[b], isem[b])

        def wait_idx_and_localize(i, b):
            @pl.when(ok(i))
            def _():
                pltpu.make_async_copy(src_hbm.at[0], sidx[b], isem[b]).wait()
                pltpu.make_async_copy(dst_hbm.at[0], didx[b], isem[b]).wait()
                for j in range(CHK // 16):
                    v = didx[b][pl.ds(j * 16, 16)] - base
                    m = (v >= 0) & (v < HNP)
                    dump = (jnp.zeros((16,), jnp.int32)
                            + (HNP + sid * 4 + (j % 4)))
                    didx[b][pl.ds(j * 16, 16)] = jnp.where(m, v, dump)

        def issue_gather(i, b):
            @pl.when(ok(i))
            def _():
                pltpu.async_copy(table_hbm.at[sidx[b]], rows[b], gsem[b])

        def wait_gather(i, b):
            @pl.when(ok(i))
            def _():
                pltpu.make_async_copy(table_hbm.at[sidx[b]], rows[b],
                                      gsem[b]).wait()

        def issue_scatter(i, b):
            @pl.when(ok(i))
            def _():
                pltpu.async_copy(rows[b], acc.at[didx[b]], ssem[b], add=True)

        def wait_scatter(i, b):
            @pl.when(ok(i))
            def _():
                pltpu.make_async_copy(rows[b], acc.at[didx[b]], ssem[b]).wait()

        issue_idx(jnp.int32(0), 0)

        @pl.loop(0, NITER, step=NBUF)
        def _steady(k0):
            for b in range(NBUF):
                i = k0 + b
                wait_scatter(i - 4, (b + 1) % NBUF)
                issue_idx(i + 1, (b + 1) % NBUF)
                wait_idx_and_localize(i, b)
                issue_gather(i, b)
                wait_gather(i - 2, (b - 2) % NBUF)
                issue_scatter(i - 2, (b - 2) % NBUF)

        for j in (NITER - 2, NITER - 1):
            wait_gather(jnp.int32(j), j % NBUF)
            issue_scatter(jnp.int32(j), j % NBUF)
        for j in range(NITER - 4, NITER):
            wait_scatter(jnp.int32(j), j % NBUF)

        plsc.subcore_barrier()
        row0 = sid * (HNP // NS)
        pltpu.sync_copy(acc.at[pl.ds(row0, HNP // NS), :],
                        out_hbm.at[pl.ds(base + row0, HNP // NS), :])

    return k(table, srcr, dstr)


def _mm1_body(x_ref, w_ref, b_ref, o_ref):
    o_ref[...] = (
        jnp.dot(x_ref[...], w_ref[...], preferred_element_type=jnp.float32)
        + b_ref[...]
    )


def _layer2_body(sp_ref, t1_ref, b_ref, w_ref, t2_ref, deg_ref):
    s = sp_ref[...]
    deg = s[:, 100:101]
    s = s[:, :112]
    t1 = t1_ref[:, :112]
    mean = jnp.where(deg > 0, s / jnp.maximum(deg, 1.0), t1)
    h1 = jnp.maximum(mean + b_ref[...], 0.0)
    t2_ref[...] = jnp.dot(h1, w_ref[...], preferred_element_type=jnp.float32)
    deg_ref[...] = deg


def _head_body(sp_ref, t2_ref, deg_ref, b2_ref, self_ref, wf1_ref, bf1_ref,
               wf2_ref, bf2_ref, o_ref, acc_ref):
    i = pl.program_id(0)
    s = sp_ref[:, :32]
    deg = deg_ref[...]
    mean = jnp.where(deg > 0, s / jnp.maximum(deg, 1.0), t2_ref[:, :32])
    h2 = jnp.maximum(mean + b2_ref[...], 0.0)
    csum = jnp.sum(h2, axis=0, keepdims=True)

    @pl.when(i == 0)
    def _():
        acc_ref[0:1, 0:32] = csum

    @pl.when(i > 0)
    def _():
        acc_ref[0:1, 0:32] += csum

    @pl.when(i == pl.num_programs(0) - 1)
    def _():
        hg = acc_ref[0:1, 0:20] * (1.0 / N_NODES)
        fused = jnp.concatenate([hg, self_ref[...]], axis=1)
        o1 = jnp.maximum(
            jnp.dot(fused, wf1_ref[...], preferred_element_type=jnp.float32)
            + bf1_ref[...], 0.0)
        o_ref[...] = (
            jnp.dot(o1, wf2_ref[...], preferred_element_type=jnp.float32)
            + bf2_ref[...]
        )


def kernel(node_feat, edge_index, self_feat, W1, b1, W2, b2, Wf1, bf1, Wf2, bf2):
    src = edge_index[0].reshape(NCHUNK, CHK)
    dst = edge_index[1].reshape(NCHUNK, CHK)

    W1Tp = jnp.zeros((128, D1P), jnp.float32).at[:, :100].set(W1.T)
    e1 = jnp.zeros((1, D1P), jnp.float32).at[0, 100].set(1.0)
    b1p = jnp.zeros((1, 112), jnp.float32).at[0, :100].set(b1)
    W2Tp = jnp.zeros((112, D2P), jnp.float32).at[:100, :20].set(W2.T)
    b2p = jnp.zeros((1, 32), jnp.float32).at[0, :20].set(b2)
    Wf1T = Wf1.T
    bf1r = bf1.reshape(1, 10)
    Wf2T = Wf2.T
    bf2r = bf2.reshape(1, 8)

    ngrid = N_NODES // BLK

    t1 = pl.pallas_call(
        _mm1_body,
        grid=(ngrid,),
        in_specs=[
            pl.BlockSpec((BLK, 128), lambda i: (i, 0)),
            pl.BlockSpec((128, D1P), lambda i: (0, 0)),
            pl.BlockSpec((1, D1P), lambda i: (0, 0)),
        ],
        out_specs=pl.BlockSpec((BLK, D1P), lambda i: (i, 0)),
        out_shape=jax.ShapeDtypeStruct((N_NODES, D1P), jnp.float32),
    )(node_feat, W1Tp, e1)

    s1 = _seg_sum_sc(t1, src, dst)

    t2, deg = pl.pallas_call(
        _layer2_body,
        grid=(ngrid,),
        in_specs=[
            pl.BlockSpec((BLK, D1P), lambda i: (i, 0)),
            pl.BlockSpec((BLK, D1P), lambda i: (i, 0)),
            pl.BlockSpec((1, 112), lambda i: (0, 0)),
            pl.BlockSpec((112, D2P), lambda i: (0, 0)),
        ],
        out_specs=[
            pl.BlockSpec((BLK, D2P), lambda i: (i, 0)),
            pl.BlockSpec((BLK, 1), lambda i: (i, 0)),
        ],
        out_shape=[
            jax.ShapeDtypeStruct((N_NODES, D2P), jnp.float32),
            jax.ShapeDtypeStruct((N_NODES, 1), jnp.float32),
        ],
    )(s1, t1, b1p, W2Tp)

    s2 = _seg_sum_sc(t2, src, dst)

    out = pl.pallas_call(
        _head_body,
        grid=(ngrid,),
        in_specs=[
            pl.BlockSpec((BLK, D2P), lambda i: (i, 0)),
            pl.BlockSpec((BLK, D2P), lambda i: (i, 0)),
            pl.BlockSpec((BLK, 1), lambda i: (i, 0)),
            pl.BlockSpec((1, 32), lambda i: (0, 0)),
            pl.BlockSpec((1, 16), lambda i: (0, 0)),
            pl.BlockSpec((36, 10), lambda i: (0, 0)),
            pl.BlockSpec((1, 10), lambda i: (0, 0)),
            pl.BlockSpec((10, 8), lambda i: (0, 0)),
            pl.BlockSpec((1, 8), lambda i: (0, 0)),
        ],
        out_specs=pl.BlockSpec((1, 8), lambda i: (0, 0)),
        out_shape=jax.ShapeDtypeStruct((1, 8), jnp.float32),
        scratch_shapes=[pltpu.VMEM((8, 128), jnp.float32)],
    )(s2, t2, deg, b2p, self_feat, Wf1T, bf1r, Wf2T, bf2r)

    return out
